# Initial kernel scaffold; baseline (speedup 1.0000x reference)
import jax
import jax.numpy as jnp
from jax.experimental import pallas as pl


def kernel(x, edge_index, W1, al1, ar1, b1, W2, al2, ar2, b2, W3, al3, ar3, b3, Wb, bb):
    raise NotImplementedError("write your pallas kernel here")


# speedup vs baseline: 1.0008x; 1.0008x over previous
import jax
import jax.numpy as jnp
from jax.experimental import pallas as pl
from jax.experimental.pallas import tpu as pltpu

N = 10000
N_HID = 128
NUM_USERS = 9500
NUM_MODELS = 100
NUM_SERVERS = 400


def _gat_layer(x, src, dst, W, a_l, a_r, bias):
    n = x.shape[0]
    h = x @ W
    e_l = h @ a_l
    e_r = h @ a_r
    edge_attn = jax.nn.leaky_relu(e_l[dst] + e_r[src], negative_slope=0.2)
    edge_max = jax.ops.segment_max(edge_attn, dst, num_segments=n)
    edge_attn = jnp.exp(edge_attn - edge_max[dst])
    edge_sum = jax.ops.segment_sum(edge_attn, dst, num_segments=n)
    edge_attn = edge_attn / (edge_sum[dst] + 1e-10)
    messages = h[src] * edge_attn[:, None]
    out = jax.ops.segment_sum(messages, dst, num_segments=n)
    return out + bias


def _score_kernel(m_ref, s_ref, wb_ref, bb_ref, out_ref):
    m = m_ref[...]
    s = s_ref[...]
    wb = wb_ref[...]
    proj = jnp.dot(m, wb, preferred_element_type=jnp.float32)
    out_ref[...] = jnp.dot(proj, s.T, preferred_element_type=jnp.float32) + bb_ref[0]


def _scores(m_emb, s_emb, Wb, bb):
    return pl.pallas_call(
        _score_kernel,
        out_shape=jax.ShapeDtypeStruct((NUM_MODELS, NUM_SERVERS), jnp.float32),
    )(m_emb, s_emb, Wb, bb)


def kernel(x, edge_index, W1, al1, ar1, b1, W2, al2, ar2, b2, W3, al3, ar3, b3, Wb, bb):
    src = edge_index[0]
    dst = edge_index[1]
    h = jax.nn.relu(_gat_layer(x, src, dst, W1, al1, ar1, b1))
    h = jax.nn.relu(_gat_layer(h, src, dst, W2, al2, ar2, b2))
    h = _gat_layer(h, src, dst, W3, al3, ar3, b3)
    m_emb = h[NUM_USERS:NUM_USERS + NUM_MODELS]
    s_emb = h[NUM_USERS + NUM_MODELS:NUM_USERS + NUM_MODELS + NUM_SERVERS]
    return _scores(m_emb, s_emb, Wb, bb)


# speedup vs baseline: 12.2664x; 12.2571x over previous
import dataclasses
import functools

import jax
import jax.numpy as jnp
from jax import lax
from jax.experimental import pallas as pl
from jax.experimental.pallas import tpu as pltpu
from jax.experimental.pallas import tpu_sc as plsc

N = 10000
N_PAD = 10240
D = 128
E = 320000
NUM_USERS = 9500
NUM_MODELS = 100
NUM_SERVERS = 400

CHUNK = 128
CPW = 158
E_PAD = 16 * CPW * CHUNK
HALF = N_PAD // 2
ACC_ROWS = 5128
OUT_PER_SUB = HALF // 16
NROWS = N_PAD // 128


def _tc_layer_body(acc_ref, d0_ref, d1_ref, b_ref, flag_ref, w_ref, al_ref,
                   ar_ref, h_ref, el_ref, er_ref):
    den = d0_ref[...] + d1_ref[...] + 1e-10
    xin = acc_ref[...] / den + b_ref[0, :]
    xin = jnp.where(flag_ref[0, :] > 0.0, jnp.maximum(xin, 0.0), xin)
    h = jnp.dot(xin, w_ref[...], preferred_element_type=jnp.float32, precision=lax.Precision.HIGHEST)
    h_ref[...] = h
    el_ref[0, 0, :] = jnp.dot(h, al_ref[0, :], preferred_element_type=jnp.float32, precision=lax.Precision.HIGHEST)
    er_ref[0, 0, :] = jnp.dot(h, ar_ref[0, :], preferred_element_type=jnp.float32, precision=lax.Precision.HIGHEST)


_BLK = 128
_GRID = N_PAD // _BLK

_h_spec = pl.BlockSpec((_BLK, D), lambda i: (i, 0))
_vec_out_spec = pl.BlockSpec((1, 1, _BLK), lambda i: (i, 0, 0))
_w_spec = pl.BlockSpec((D, D), lambda i: (0, 0))
_a_spec = pl.BlockSpec((1, D), lambda i: (0, 0))
_den_spec = pl.BlockSpec((_BLK, 1), lambda i: (i, 0))

_layer_out_shapes = (
    jax.ShapeDtypeStruct((N_PAD, D), jnp.float32),
    jax.ShapeDtypeStruct((_GRID, 1, _BLK), jnp.float32),
    jax.ShapeDtypeStruct((_GRID, 1, _BLK), jnp.float32),
)
_layer_out_specs = (_h_spec, _vec_out_spec, _vec_out_spec)


def _tc_layer(acc, d0, d1, b, flag, W, al, ar):
    return pl.pallas_call(
        _tc_layer_body,
        grid=(_GRID,),
        in_specs=[_h_spec, _den_spec, _den_spec, _a_spec, _a_spec, _w_spec,
                  _a_spec, _a_spec],
        out_specs=_layer_out_specs,
        out_shape=_layer_out_shapes,
    )(acc, d0, d1, b.reshape(1, D), flag.reshape(1, D), W,
      al.reshape(1, D), ar.reshape(1, D))


def _tc_final_body(acc_ref, d0_ref, d1_ref, b_ref, wb_ref, bb_ref, out_ref):
    den = d0_ref[...] + d1_ref[...] + 1e-10
    emb = acc_ref[...] / den + b_ref[0, :]
    m = emb[:NUM_MODELS]
    s = emb[NUM_MODELS:]
    proj = jnp.dot(m, wb_ref[...], preferred_element_type=jnp.float32, precision=lax.Precision.HIGHEST)
    scores = lax.dot_general(proj, s, (((1,), (1,)), ((), ())),
                             preferred_element_type=jnp.float32, precision=lax.Precision.HIGHEST)
    out_ref[...] = scores + bb_ref[0, 0]


def _tc_final(accs, d0s, d1s, b3, Wb, bb):
    return pl.pallas_call(
        _tc_final_body,
        out_shape=jax.ShapeDtypeStruct((NUM_MODELS, NUM_SERVERS), jnp.float32),
    )(accs, d0s, d1s, b3.reshape(1, D), Wb, bb.reshape(1, 1))


def _sc_compiler_params():
    cp = pltpu.CompilerParams()
    if "needs_layout_passes" in pltpu.CompilerParams.__dataclass_fields__:
        cp = dataclasses.replace(cp, needs_layout_passes=False)
    return cp


def _split_idx(i16):
    return [lax.shift_right_logical(i16, 7), lax.bitwise_and(i16, 127)]


def _sc_edge_pass(h, el, er, dst3, src3, mvec, zrows, iota80):
    mesh = plsc.VectorSubcoreMesh(core_axis_name="c", subcore_axis_name="s")

    @functools.partial(
        pl.kernel,
        out_type=(
            jax.ShapeDtypeStruct((2, HALF, D), jnp.float32),
            jax.ShapeDtypeStruct((2, NROWS, 128), jnp.float32),
        ),
        mesh=mesh,
        compiler_params=_sc_compiler_params(),
        scratch_types=[
            pltpu.VMEM((NROWS, 128), jnp.float32),
            pltpu.VMEM((NROWS, 128), jnp.float32),
            pltpu.VMEM((CPW, CHUNK), jnp.int32),
            pltpu.VMEM((CPW, CHUNK), jnp.int32),
            pltpu.VMEM((CHUNK,), jnp.int32),
            pltpu.VMEM((CHUNK, D), jnp.float32),
            pltpu.VMEM((CHUNK,), jnp.float32),
            pltpu.VMEM((16,), jnp.float32),
            pltpu.VMEM((NROWS, 128), jnp.float32),
            pltpu.VMEM((NROWS,), jnp.int32),
            pltpu.VMEM_SHARED((ACC_ROWS, D), jnp.float32),
            pltpu.VMEM_SHARED((NROWS, 128), jnp.float32),
            pltpu.SemaphoreType.DMA,
        ],
    )
    def sc_kernel(h_hbm, el_hbm, er_hbm, dst_hbm, src_hbm, m_hbm, zr_hbm,
                  iota_hbm, out_hbm, den_hbm, el_v, er_v, dst_v, src_v,
                  dloc_v, rows_v, p_v, m_v, den_v, iota_v, acc_sh, den_sh,
                  gsem):
        c = lax.axis_index("c")
        s = lax.axis_index("s")

        pltpu.sync_copy(el_hbm, el_v)
        pltpu.sync_copy(er_hbm, er_v)
        pltpu.sync_copy(dst_hbm.at[s], dst_v)
        pltpu.sync_copy(src_hbm.at[s], src_v)
        pltpu.sync_copy(m_hbm, m_v)
        pltpu.sync_copy(zr_hbm.at[pl.ds(0, NROWS)], den_v)
        pltpu.sync_copy(iota_hbm, iota_v)
        pltpu.sync_copy(zr_hbm.at[pl.ds(0, OUT_PER_SUB)],
                        acc_sh.at[pl.ds(s * OUT_PER_SUB, OUT_PER_SUB)])

        @pl.when(s == 0)
        def _zero_den():
            pltpu.sync_copy(zr_hbm.at[pl.ds(0, NROWS)], den_sh)
            pltpu.sync_copy(zr_hbm.at[pl.ds(0, ACC_ROWS - HALF)],
                            acc_sh.at[pl.ds(HALF, ACC_ROWS - HALF)])

        plsc.subcore_barrier()

        mv = m_v[...]
        base = c * HALF

        @pl.loop(0, CPW)
        def _chunk(j):
            cp = pltpu.async_copy(h_hbm.at[src_v.at[j]], rows_v, gsem)

            @pl.loop(0, CHUNK // 16)
            def _pblk(k):
                d16 = dst_v[j, pl.ds(k * 16, 16)]
                s16 = src_v[j, pl.ds(k * 16, 16)]
                z = (plsc.load_gather(el_v, _split_idx(d16))
                     + plsc.load_gather(er_v, _split_idx(s16)))
                z = jnp.maximum(z, 0.2 * z)
                p16 = jnp.exp(z - mv)
                p_v[pl.ds(k * 16, 16)] = p16
                dl = d16 - base
                valid = jnp.logical_and(dl >= 0, dl < HALF)
                dloc_v[pl.ds(k * 16, 16)] = jnp.where(dl >= 0,
                                                      jnp.where(dl < HALF, dl, HALF),
                                                      HALF)
                plsc.addupdate_scatter(den_v, _split_idx(d16),
                                       jnp.where(valid, p16, 0.0))

            cp.wait()

            @pl.loop(0, CHUNK)
            def _row(r):
                pr = plsc.load_gather(p_v, [jnp.full((16,), r, jnp.int32)])
                for kk in range(D // 16):
                    rows_v[r, pl.ds(kk * 16, 16)] = (
                        rows_v[r, pl.ds(kk * 16, 16)] * pr)

            pltpu.sync_copy(rows_v, acc_sh.at[dloc_v], add=True)

        pltpu.sync_copy(den_v, den_sh.at[iota_v], add=True)

        plsc.subcore_barrier()
        sl = pl.ds(s * OUT_PER_SUB, OUT_PER_SUB)
        pltpu.sync_copy(acc_sh.at[sl], out_hbm.at[c].at[sl])

        @pl.when(s < NROWS // 8)
        def _den_out():
            dsl = pl.ds(s * 8, 8)
            pltpu.sync_copy(den_sh.at[dsl], den_hbm.at[c].at[dsl])

    return sc_kernel(h, el, er, dst3, src3, mvec, zrows, iota80)


def _global_shift(EL, ER):
    m0 = jnp.max(EL) + jnp.max(ER)
    m0 = jnp.maximum(m0, 0.2 * m0)
    return jnp.maximum(m0, 0.0)


def kernel(x, edge_index, W1, al1, ar1, b1, W2, al2, ar2, b2, W3, al3, ar3,
           b3, Wb, bb):
    src = edge_index[0]
    dst = edge_index[1]
    srcp = jnp.concatenate(
        [src, jnp.zeros((E_PAD - E,), jnp.int32)]).reshape(16, CPW, CHUNK)
    dstp = jnp.concatenate(
        [dst, jnp.full((E_PAD - E,), N, jnp.int32)]).reshape(16, CPW, CHUNK)
    zrows = jnp.zeros((OUT_PER_SUB, D), jnp.float32)
    iota80 = jnp.arange(NROWS, dtype=jnp.int32)
    xp = jnp.pad(x, ((0, N_PAD - N), (0, 0)))

    def sc_pass(H, EL, ER):
        mvec = jnp.full((16,), _global_shift(EL, ER), jnp.float32)
        P, DEN = _sc_edge_pass(H, EL.reshape(NROWS, 128), ER.reshape(NROWS, 128),
                               dstp, srcp, mvec, zrows, iota80)
        return P.reshape(N_PAD, D), DEN.reshape(2, N_PAD, 1)

    Ws = jnp.stack([W1, W2, W3])
    als = jnp.stack([al1, al2, al3])
    ars = jnp.stack([ar1, ar2, ar3])
    bs = jnp.stack([jnp.zeros_like(b1), b1, b2])
    flags = jnp.stack([jnp.zeros((D,), jnp.float32),
                       jnp.ones((D,), jnp.float32),
                       jnp.ones((D,), jnp.float32)])

    def body(carry, xs):
        ACC, D0, D1 = carry
        W, al, ar, b, flag = xs
        H, EL, ER = _tc_layer(ACC, D0, D1, b, flag, W, al, ar)
        ACC, DEN = sc_pass(H, EL, ER)
        return (ACC, DEN[0], DEN[1]), None

    init = (xp, jnp.ones((N_PAD, 1), jnp.float32),
            jnp.zeros((N_PAD, 1), jnp.float32))
    (ACC, D0, D1), _ = lax.scan(body, init, (Ws, als, ars, bs, flags))

    lo, hi = NUM_USERS, NUM_USERS + NUM_MODELS + NUM_SERVERS
    return _tc_final(ACC[lo:hi], D0[lo:hi], D1[lo:hi], b3, Wb, bb)


# speedup vs baseline: 15.9190x; 1.2978x over previous
import dataclasses
import functools

import jax
import jax.numpy as jnp
from jax import lax
from jax.experimental import pallas as pl
from jax.experimental.pallas import tpu as pltpu
from jax.experimental.pallas import tpu_sc as plsc

N = 10000
N_PAD = 10240
D = 128
E = 320000
NUM_USERS = 9500
NUM_MODELS = 100
NUM_SERVERS = 400

CHUNK = 64
CPW = 314
IDXR = 158
E_PAD = 16 * CPW * CHUNK
HALF = N_PAD // 2
ACC_ROWS = HALF
OUT_PER_SUB = HALF // 16
NROWS = N_PAD // 128


def _tc_layer_body(acc_ref, d0_ref, d1_ref, b_ref, flag_ref, w_ref, al_ref,
                   ar_ref, h_ref, el_ref, er_ref):
    den = d0_ref[...] + d1_ref[...] + 1e-10
    xin = acc_ref[...] / den + b_ref[0, :]
    xin = jnp.where(flag_ref[0, :] > 0.0, jnp.maximum(xin, 0.0), xin)
    h = jnp.dot(xin, w_ref[...], preferred_element_type=jnp.float32, precision=lax.Precision.HIGHEST)
    h_ref[...] = h
    el_ref[0, 0, :] = jnp.dot(h, al_ref[0, :], preferred_element_type=jnp.float32, precision=lax.Precision.HIGHEST)
    er_ref[0, 0, :] = jnp.dot(h, ar_ref[0, :], preferred_element_type=jnp.float32, precision=lax.Precision.HIGHEST)


_BLK = 128
_GRID = N_PAD // _BLK

_h_spec = pl.BlockSpec((_BLK, D), lambda i: (i, 0))
_vec_out_spec = pl.BlockSpec((1, 1, _BLK), lambda i: (i, 0, 0))
_w_spec = pl.BlockSpec((D, D), lambda i: (0, 0))
_a_spec = pl.BlockSpec((1, D), lambda i: (0, 0))
_den_spec = pl.BlockSpec((_BLK, 1), lambda i: (i, 0))

_layer_out_shapes = (
    jax.ShapeDtypeStruct((N_PAD, D), jnp.float32),
    jax.ShapeDtypeStruct((_GRID, 1, _BLK), jnp.float32),
    jax.ShapeDtypeStruct((_GRID, 1, _BLK), jnp.float32),
)
_layer_out_specs = (_h_spec, _vec_out_spec, _vec_out_spec)


def _tc_layer(acc, d0, d1, b, flag, W, al, ar):
    return pl.pallas_call(
        _tc_layer_body,
        grid=(_GRID,),
        in_specs=[_h_spec, _den_spec, _den_spec, _a_spec, _a_spec, _w_spec,
                  _a_spec, _a_spec],
        out_specs=_layer_out_specs,
        out_shape=_layer_out_shapes,
    )(acc, d0, d1, b.reshape(1, D), flag.reshape(1, D), W,
      al.reshape(1, D), ar.reshape(1, D))


def _tc_final_body(acc_ref, d0_ref, d1_ref, b_ref, wb_ref, bb_ref, out_ref):
    den = d0_ref[...] + d1_ref[...] + 1e-10
    emb = acc_ref[...] / den + b_ref[0, :]
    m = emb[:NUM_MODELS]
    s = emb[NUM_MODELS:]
    proj = jnp.dot(m, wb_ref[...], preferred_element_type=jnp.float32, precision=lax.Precision.HIGHEST)
    scores = lax.dot_general(proj, s, (((1,), (1,)), ((), ())),
                             preferred_element_type=jnp.float32, precision=lax.Precision.HIGHEST)
    out_ref[...] = scores + bb_ref[0, 0]


def _tc_final(accs, d0s, d1s, b3, Wb, bb):
    return pl.pallas_call(
        _tc_final_body,
        out_shape=jax.ShapeDtypeStruct((NUM_MODELS, NUM_SERVERS), jnp.float32),
    )(accs, d0s, d1s, b3.reshape(1, D), Wb, bb.reshape(1, 1))


def _sc_compiler_params():
    cp = pltpu.CompilerParams()
    if "needs_layout_passes" in pltpu.CompilerParams.__dataclass_fields__:
        cp = dataclasses.replace(cp, needs_layout_passes=False)
    return cp


def _split_idx(i16):
    return [lax.shift_right_logical(i16, 7), lax.bitwise_and(i16, 127)]


def _sc_edge_pass(h, el, er, dst3, src3, rvec, zrows):
    mesh = plsc.VectorSubcoreMesh(core_axis_name="c", subcore_axis_name="s")

    @functools.partial(
        pl.kernel,
        out_type=(
            jax.ShapeDtypeStruct((2, HALF, D), jnp.float32),
            jax.ShapeDtypeStruct((2, 16, NROWS, 128), jnp.float32),
        ),
        mesh=mesh,
        compiler_params=_sc_compiler_params(),
        scratch_types=[
            pltpu.VMEM((NROWS, 128), jnp.float32),
            pltpu.VMEM((NROWS, 128), jnp.float32),
            pltpu.VMEM((IDXR, 128), jnp.int32),
            pltpu.VMEM((IDXR, 128), jnp.int32),
            pltpu.VMEM((CHUNK,), jnp.int32),
            pltpu.VMEM((2 * CHUNK, D), jnp.float32),
            pltpu.VMEM((CHUNK,), jnp.float32),
            pltpu.VMEM((16,), jnp.float32),
            pltpu.VMEM((NROWS, 128), jnp.float32),
            pltpu.VMEM_SHARED((ACC_ROWS, D), jnp.float32),
            pltpu.SemaphoreType.DMA((2,)),
        ],
    )
    def sc_kernel(h_hbm, el_hbm, er_hbm, dst_hbm, src_hbm, r_hbm, zr_hbm,
                  out_hbm, den_hbm, el_v, er_v, dst_v, src_v,
                  dloc_v, rows_v, p_v, r_v, den_v, acc_sh, gsem):
        c = lax.axis_index("c")
        s = lax.axis_index("s")

        pltpu.sync_copy(el_hbm, el_v)
        pltpu.sync_copy(er_hbm, er_v)
        pltpu.sync_copy(dst_hbm.at[s], dst_v)
        pltpu.sync_copy(src_hbm.at[s], src_v)
        pltpu.sync_copy(r_hbm, r_v)
        pltpu.sync_copy(zr_hbm.at[pl.ds(0, NROWS)], den_v)
        pltpu.sync_copy(zr_hbm, acc_sh.at[pl.ds(s * OUT_PER_SUB, OUT_PER_SUB)])

        plsc.subcore_barrier()

        rv = r_v[...]
        base = c * HALF

        pltpu.async_copy(h_hbm.at[src_v.at[0].at[pl.ds(0, CHUNK)]],
                         rows_v.at[pl.ds(0, CHUNK)], gsem.at[0])

        @pl.loop(0, CPW)
        def _chunk(j):
            jr = j >> 1
            off = (j & 1) * CHUNK
            jr2 = (j + 1) >> 1
            noff = CHUNK - off

            @pl.loop(0, CHUNK // 16)
            def _pblk(k):
                d16 = dst_v[jr, pl.ds(off + k * 16, 16)]
                s16 = src_v[jr, pl.ds(off + k * 16, 16)]
                a16 = plsc.load_gather(el_v, _split_idx(d16))
                z = a16 + plsc.load_gather(er_v, _split_idx(s16))
                z = jnp.maximum(z, 0.2 * z)
                t16 = a16 + rv
                sh16 = jnp.maximum(jnp.maximum(t16, 0.2 * t16), 0.0)
                dl = d16 - base
                valid = jnp.logical_and(dl >= 0, dl < HALF)
                p16 = jnp.where(valid, jnp.exp(z - sh16), 0.0)
                p_v[pl.ds(k * 16, 16)] = p16
                dloc_v[pl.ds(k * 16, 16)] = jnp.where(valid, dl, 0)
                plsc.addupdate_scatter(den_v, _split_idx(d16), p16)

            pltpu.make_async_copy(h_hbm.at[src_v.at[jr].at[pl.ds(off, CHUNK)]],
                                  rows_v.at[pl.ds(off, CHUNK)],
                                  gsem.at[j & 1]).wait()
            pltpu.async_copy(h_hbm.at[src_v.at[jr2].at[pl.ds(noff, CHUNK)]],
                             rows_v.at[pl.ds(noff, CHUNK)],
                             gsem.at[(j + 1) & 1])

            @pl.loop(0, CHUNK)
            def _row(r):
                pr = plsc.load_gather(p_v, [jnp.full((16,), r, jnp.int32)])
                for kk in range(D // 16):
                    rows_v[off + r, pl.ds(kk * 16, 16)] = (
                        rows_v[off + r, pl.ds(kk * 16, 16)] * pr)

            pltpu.sync_copy(rows_v.at[pl.ds(off, CHUNK)],
                            acc_sh.at[dloc_v], add=True)

        pltpu.make_async_copy(
            h_hbm.at[src_v.at[CPW >> 1].at[pl.ds(0, CHUNK)]],
            rows_v.at[pl.ds(0, CHUNK)], gsem.at[CPW & 1]).wait()

        pltpu.sync_copy(den_v, den_hbm.at[c].at[s])

        plsc.subcore_barrier()
        sl = pl.ds(s * OUT_PER_SUB, OUT_PER_SUB)
        pltpu.sync_copy(acc_sh.at[sl], out_hbm.at[c].at[sl])

    return sc_kernel(h, el, er, dst3, src3, rvec, zrows)


def kernel(x, edge_index, W1, al1, ar1, b1, W2, al2, ar2, b2, W3, al3, ar3,
           b3, Wb, bb):
    src = edge_index[0]
    dst = edge_index[1]
    srcp = jnp.concatenate(
        [src, jnp.zeros((E_PAD - E,), jnp.int32)]).reshape(16, IDXR - 1, 128)
    srcp = jnp.pad(srcp, ((0, 0), (0, 1), (0, 0)))
    dstp = jnp.concatenate(
        [dst, jnp.full((E_PAD - E,), N, jnp.int32)]).reshape(16, IDXR - 1, 128)
    dstp = jnp.pad(dstp, ((0, 0), (0, 1), (0, 0)), constant_values=N)
    zrows = jnp.zeros((OUT_PER_SUB, D), jnp.float32)
    xp = jnp.pad(x, ((0, N_PAD - N), (0, 0)))

    def sc_pass(H, EL, ER):
        rvec = jnp.full((16,), jnp.max(ER), jnp.float32)
        P, DEN = _sc_edge_pass(H, EL.reshape(NROWS, 128), ER.reshape(NROWS, 128),
                               dstp, srcp, rvec, zrows)
        return P.reshape(N_PAD, D), DEN.sum(axis=1).reshape(2, N_PAD, 1)

    Ws = jnp.stack([W1, W2, W3])
    als = jnp.stack([al1, al2, al3])
    ars = jnp.stack([ar1, ar2, ar3])
    bs = jnp.stack([jnp.zeros_like(b1), b1, b2])
    flags = jnp.stack([jnp.zeros((D,), jnp.float32),
                       jnp.ones((D,), jnp.float32),
                       jnp.ones((D,), jnp.float32)])

    def body(carry, xs):
        ACC, D0, D1 = carry
        W, al, ar, b, flag = xs
        H, EL, ER = _tc_layer(ACC, D0, D1, b, flag, W, al, ar)
        ACC, DEN = sc_pass(H, EL, ER)
        return (ACC, DEN[0], DEN[1]), None

    init = (xp, jnp.ones((N_PAD, 1), jnp.float32),
            jnp.zeros((N_PAD, 1), jnp.float32))
    (ACC, D0, D1), _ = lax.scan(body, init, (Ws, als, ars, bs, flags))

    lo, hi = NUM_USERS, NUM_USERS + NUM_MODELS + NUM_SERVERS
    return _tc_final(ACC[lo:hi], D0[lo:hi], D1[lo:hi], b3, Wb, bb)


# speedup vs baseline: 16.4294x; 1.0321x over previous
import dataclasses
import functools

import jax
import jax.numpy as jnp
from jax import lax
from jax.experimental import pallas as pl
from jax.experimental.pallas import tpu as pltpu
from jax.experimental.pallas import tpu_sc as plsc

N = 10000
N_PAD = 10240
D = 128
E = 320000
NUM_USERS = 9500
NUM_MODELS = 100
NUM_SERVERS = 400

CHUNK = 64
CPW = 314
IDXR = 158
E_PAD = 16 * CPW * CHUNK
HALF = N_PAD // 2
ACC_ROWS = HALF
OUT_PER_SUB = HALF // 16
NROWS = N_PAD // 128


def _tc_layer_body(acc_ref, d0_ref, d1_ref, b_ref, flag_ref, w_ref, al_ref,
                   ar_ref, h_ref, el_ref, er_ref):
    den = d0_ref[...] + d1_ref[...] + 1e-10
    xin = acc_ref[...] / den + b_ref[0, :]
    xin = jnp.where(flag_ref[0, :] > 0.0, jnp.maximum(xin, 0.0), xin)
    h = jnp.dot(xin, w_ref[...], preferred_element_type=jnp.float32, precision=lax.Precision.HIGHEST)
    h_ref[...] = h
    el_ref[0, 0, :] = jnp.dot(h, al_ref[0, :], preferred_element_type=jnp.float32, precision=lax.Precision.HIGHEST)
    er_ref[0, 0, :] = jnp.dot(h, ar_ref[0, :], preferred_element_type=jnp.float32, precision=lax.Precision.HIGHEST)


_BLK = 128
_GRID = N_PAD // _BLK

_h_spec = pl.BlockSpec((_BLK, D), lambda i: (i, 0))
_vec_out_spec = pl.BlockSpec((1, 1, _BLK), lambda i: (i, 0, 0))
_w_spec = pl.BlockSpec((D, D), lambda i: (0, 0))
_a_spec = pl.BlockSpec((1, D), lambda i: (0, 0))
_den_spec = pl.BlockSpec((_BLK, 1), lambda i: (i, 0))

_layer_out_shapes = (
    jax.ShapeDtypeStruct((N_PAD, D), jnp.float32),
    jax.ShapeDtypeStruct((_GRID, 1, _BLK), jnp.float32),
    jax.ShapeDtypeStruct((_GRID, 1, _BLK), jnp.float32),
)
_layer_out_specs = (_h_spec, _vec_out_spec, _vec_out_spec)


def _tc_layer(acc, d0, d1, b, flag, W, al, ar):
    return pl.pallas_call(
        _tc_layer_body,
        grid=(_GRID,),
        in_specs=[_h_spec, _den_spec, _den_spec, _a_spec, _a_spec, _w_spec,
                  _a_spec, _a_spec],
        out_specs=_layer_out_specs,
        out_shape=_layer_out_shapes,
    )(acc, d0, d1, b.reshape(1, D), flag.reshape(1, D), W,
      al.reshape(1, D), ar.reshape(1, D))


def _tc_final_body(acc_ref, d0_ref, d1_ref, b_ref, wb_ref, bb_ref, out_ref):
    den = d0_ref[...] + d1_ref[...] + 1e-10
    emb = acc_ref[...] / den + b_ref[0, :]
    m = emb[:NUM_MODELS]
    s = emb[NUM_MODELS:]
    proj = jnp.dot(m, wb_ref[...], preferred_element_type=jnp.float32, precision=lax.Precision.HIGHEST)
    scores = lax.dot_general(proj, s, (((1,), (1,)), ((), ())),
                             preferred_element_type=jnp.float32, precision=lax.Precision.HIGHEST)
    out_ref[...] = scores + bb_ref[0, 0]


def _tc_final(accs, d0s, d1s, b3, Wb, bb):
    return pl.pallas_call(
        _tc_final_body,
        out_shape=jax.ShapeDtypeStruct((NUM_MODELS, NUM_SERVERS), jnp.float32),
    )(accs, d0s, d1s, b3.reshape(1, D), Wb, bb.reshape(1, 1))


def _sc_compiler_params():
    cp = pltpu.CompilerParams()
    if "needs_layout_passes" in pltpu.CompilerParams.__dataclass_fields__:
        cp = dataclasses.replace(cp, needs_layout_passes=False)
    return cp


def _split_idx(i16):
    return [lax.shift_right_logical(i16, 7), lax.bitwise_and(i16, 127)]


def _sc_edge_pass(h, el, er, dst3, src3, rvec, zrows):
    mesh = plsc.VectorSubcoreMesh(core_axis_name="c", subcore_axis_name="s")

    @functools.partial(
        pl.kernel,
        out_type=(
            jax.ShapeDtypeStruct((2, HALF, D), jnp.float32),
            jax.ShapeDtypeStruct((2, 16, NROWS, 128), jnp.float32),
        ),
        mesh=mesh,
        compiler_params=_sc_compiler_params(),
        scratch_types=[
            pltpu.VMEM((NROWS, 128), jnp.float32),
            pltpu.VMEM((NROWS, 128), jnp.float32),
            pltpu.VMEM((IDXR, 128), jnp.int32),
            pltpu.VMEM((IDXR, 128), jnp.int32),
            pltpu.VMEM((2 * CHUNK,), jnp.int32),
            pltpu.VMEM((2 * CHUNK, D), jnp.float32),
            pltpu.VMEM((CHUNK,), jnp.float32),
            pltpu.VMEM((16,), jnp.float32),
            pltpu.VMEM((NROWS, 128), jnp.float32),
            pltpu.VMEM_SHARED((ACC_ROWS, D), jnp.float32),
            pltpu.SemaphoreType.DMA((2,)),
            pltpu.SemaphoreType.DMA((2,)),
        ],
    )
    def sc_kernel(h_hbm, el_hbm, er_hbm, dst_hbm, src_hbm, r_hbm, zr_hbm,
                  out_hbm, den_hbm, el_v, er_v, dst_v, src_v,
                  dloc_v, rows_v, p_v, r_v, den_v, acc_sh, gsem, ssem):
        c = lax.axis_index("c")
        s = lax.axis_index("s")

        pltpu.sync_copy(el_hbm, el_v)
        pltpu.sync_copy(er_hbm, er_v)
        pltpu.sync_copy(dst_hbm.at[s], dst_v)
        pltpu.sync_copy(src_hbm.at[s], src_v)
        pltpu.sync_copy(r_hbm, r_v)
        pltpu.sync_copy(zr_hbm.at[pl.ds(0, NROWS)], den_v)
        pltpu.sync_copy(zr_hbm, acc_sh.at[pl.ds(s * OUT_PER_SUB, OUT_PER_SUB)])

        plsc.subcore_barrier()

        rv = r_v[...]
        base = c * HALF

        pltpu.async_copy(h_hbm.at[src_v.at[0].at[pl.ds(0, CHUNK)]],
                         rows_v.at[pl.ds(0, CHUNK)], gsem.at[0])

        @pl.loop(0, CPW)
        def _chunk(j):
            jr = j >> 1
            off = (j & 1) * CHUNK
            jr2 = (j + 1) >> 1
            noff = CHUNK - off

            @pl.loop(0, CHUNK // 16)
            def _pblk(k):
                d16 = dst_v[jr, pl.ds(off + k * 16, 16)]
                s16 = src_v[jr, pl.ds(off + k * 16, 16)]
                a16 = plsc.load_gather(el_v, _split_idx(d16))
                z = a16 + plsc.load_gather(er_v, _split_idx(s16))
                z = jnp.maximum(z, 0.2 * z)
                t16 = a16 + rv
                sh16 = jnp.maximum(jnp.maximum(t16, 0.2 * t16), 0.0)
                dl = d16 - base
                valid = jnp.logical_and(dl >= 0, dl < HALF)
                p16 = jnp.where(valid, jnp.exp(z - sh16), 0.0)
                p_v[pl.ds(k * 16, 16)] = p16
                dloc_v[pl.ds(off + k * 16, 16)] = jnp.where(valid, dl, 0)
                plsc.addupdate_scatter(den_v, _split_idx(d16), p16)

            pltpu.make_async_copy(h_hbm.at[src_v.at[jr].at[pl.ds(off, CHUNK)]],
                                  rows_v.at[pl.ds(off, CHUNK)],
                                  gsem.at[j & 1]).wait()

            @pl.when(j > 0)
            def _drain_scatter():
                pltpu.make_async_copy(rows_v.at[pl.ds(noff, CHUNK)],
                                      acc_sh.at[dloc_v.at[pl.ds(noff, CHUNK)]],
                                      ssem.at[(j + 1) & 1]).wait()

            pltpu.async_copy(h_hbm.at[src_v.at[jr2].at[pl.ds(noff, CHUNK)]],
                             rows_v.at[pl.ds(noff, CHUNK)],
                             gsem.at[(j + 1) & 1])

            @pl.loop(0, CHUNK)
            def _row(r):
                pr = plsc.load_gather(p_v, [jnp.full((16,), r, jnp.int32)])
                for kk in range(D // 16):
                    rows_v[off + r, pl.ds(kk * 16, 16)] = (
                        rows_v[off + r, pl.ds(kk * 16, 16)] * pr)

            pltpu.async_copy(rows_v.at[pl.ds(off, CHUNK)],
                             acc_sh.at[dloc_v.at[pl.ds(off, CHUNK)]],
                             ssem.at[j & 1], add=True)

        pltpu.make_async_copy(
            h_hbm.at[src_v.at[CPW >> 1].at[pl.ds(0, CHUNK)]],
            rows_v.at[pl.ds(0, CHUNK)], gsem.at[CPW & 1]).wait()
        pltpu.make_async_copy(rows_v.at[pl.ds(CHUNK, CHUNK)],
                              acc_sh.at[dloc_v.at[pl.ds(CHUNK, CHUNK)]],
                              ssem.at[1]).wait()

        pltpu.sync_copy(den_v, den_hbm.at[c].at[s])

        plsc.subcore_barrier()
        sl = pl.ds(s * OUT_PER_SUB, OUT_PER_SUB)
        pltpu.sync_copy(acc_sh.at[sl], out_hbm.at[c].at[sl])

    return sc_kernel(h, el, er, dst3, src3, rvec, zrows)


def kernel(x, edge_index, W1, al1, ar1, b1, W2, al2, ar2, b2, W3, al3, ar3,
           b3, Wb, bb):
    src = edge_index[0]
    dst = edge_index[1]
    srcp = jnp.concatenate(
        [src, jnp.zeros((E_PAD - E,), jnp.int32)]).reshape(16, IDXR - 1, 128)
    srcp = jnp.pad(srcp, ((0, 0), (0, 1), (0, 0)))
    dstp = jnp.concatenate(
        [dst, jnp.full((E_PAD - E,), N, jnp.int32)]).reshape(16, IDXR - 1, 128)
    dstp = jnp.pad(dstp, ((0, 0), (0, 1), (0, 0)), constant_values=N)
    zrows = jnp.zeros((OUT_PER_SUB, D), jnp.float32)
    xp = jnp.pad(x, ((0, N_PAD - N), (0, 0)))

    def sc_pass(H, EL, ER):
        rvec = jnp.full((16,), jnp.max(ER), jnp.float32)
        P, DEN = _sc_edge_pass(H, EL.reshape(NROWS, 128), ER.reshape(NROWS, 128),
                               dstp, srcp, rvec, zrows)
        return P.reshape(N_PAD, D), DEN.sum(axis=1).reshape(2, N_PAD, 1)

    Ws = jnp.stack([W1, W2, W3])
    als = jnp.stack([al1, al2, al3])
    ars = jnp.stack([ar1, ar2, ar3])
    bs = jnp.stack([jnp.zeros_like(b1), b1, b2])
    flags = jnp.stack([jnp.zeros((D,), jnp.float32),
                       jnp.ones((D,), jnp.float32),
                       jnp.ones((D,), jnp.float32)])

    def body(carry, xs):
        ACC, D0, D1 = carry
        W, al, ar, b, flag = xs
        H, EL, ER = _tc_layer(ACC, D0, D1, b, flag, W, al, ar)
        ACC, DEN = sc_pass(H, EL, ER)
        return (ACC, DEN[0], DEN[1]), None

    init = (xp, jnp.ones((N_PAD, 1), jnp.float32),
            jnp.zeros((N_PAD, 1), jnp.float32))
    (ACC, D0, D1), _ = lax.scan(body, init, (Ws, als, ars, bs, flags))

    lo, hi = NUM_USERS, NUM_USERS + NUM_MODELS + NUM_SERVERS
    return _tc_final(ACC[lo:hi], D0[lo:hi], D1[lo:hi], b3, Wb, bb)
